# baseline (speedup 1.0000x reference)
import functools

import jax
import jax.numpy as jnp
from jax import lax
from jax.experimental import pallas as pl
from jax.experimental.pallas import tpu as pltpu
from jax.experimental.pallas import tpu_sc as plsc

N = 10000
E = 320000
D = 128
G = 64
EPS = 1e-5

NC, NS = 2, 16
NW = NC * NS
C = 128
PAD_ROWS = 256
NA = N + PAD_ROWS
ECH = 2592
EP = ECH * C
NCHW = ECH // NW
KB = 1
NGRP = NCHW // KB
RB = 200
NRB = N // RB
NRB_PER, NRB_REM = NRB // NS, NRB % NS

_MESH = dict(core_axis_name="c", subcore_axis_name="s")


def _deg_kernel(dst_hbm, ones_hbm, z_hbm, out_hbm, idx_d, ones_v, acc):
    c = lax.axis_index("c")
    s = lax.axis_index("s")
    wid = s * NC + c
    pltpu.sync_copy(ones_hbm, ones_v)

    nblk = jnp.where(s < NRB_REM, NRB_PER + 1, NRB_PER)

    def zbody(k, _):
        b = s + NS * k
        pltpu.sync_copy(z_hbm.at[pl.ds(b * RB, RB)], acc.at[pl.ds(b * RB, RB)])
        return 0

    lax.fori_loop(0, nblk, zbody, 0)
    plsc.subcore_barrier()

    def ebody(j, _):
        base = (wid + NW * j) * C
        pltpu.sync_copy(dst_hbm.at[pl.ds(base, C)], idx_d)
        pltpu.sync_copy(ones_v, acc.at[idx_d], add=True)
        return 0

    lax.fori_loop(0, NCHW, ebody, 0)
    plsc.subcore_barrier()

    def obody(k, _):
        b = s + NS * k
        pltpu.sync_copy(acc.at[pl.ds(b * RB, RB)],
                        out_hbm.at[pl.ds(c * N + b * RB, RB)])
        return 0

    lax.fori_loop(0, nblk, obody, 0)


def _sc_degree(dst, ones_c, zeros_nd):
    k = pl.kernel(
        _deg_kernel,
        out_type=jax.ShapeDtypeStruct((NC * N, D), jnp.float32),
        mesh=plsc.VectorSubcoreMesh(**_MESH),
        scratch_types=[
            pltpu.VMEM((C,), jnp.int32),
            pltpu.VMEM((C, D), jnp.float32),
            pltpu.VMEM_SHARED((NA, D), jnp.float32),
        ],
    )
    return k(dst, ones_c, zeros_nd)


def _spmm_kernel(xs_hbm, src_hbm, dst_hbm, z_hbm, out_hbm, *scr):
    idx_s = scr[0:KB]
    idx_d = scr[KB:2 * KB]
    rows = scr[2 * KB:3 * KB]
    gsem = scr[3 * KB:4 * KB]
    acc = scr[4 * KB]

    c = lax.axis_index("c")
    s = lax.axis_index("s")
    wid = s * NC + c

    nblk = jnp.where(s < NRB_REM, NRB_PER + 1, NRB_PER)

    def zbody(k, _):
        b = s + NS * k
        pltpu.sync_copy(z_hbm.at[pl.ds(b * RB, RB)], acc.at[pl.ds(b * RB, RB)])
        return 0

    lax.fori_loop(0, nblk, zbody, 0)
    plsc.subcore_barrier()

    def load_and_gather(k, j):
        base = (wid + NW * j) * C
        pltpu.sync_copy(src_hbm.at[pl.ds(base, C)], idx_s[k])
        pltpu.sync_copy(dst_hbm.at[pl.ds(base, C)], idx_d[k])
        pltpu.async_copy(xs_hbm.at[idx_s[k]], rows[k], gsem[k])

    for k in range(KB):
        load_and_gather(k, k)

    def group(g, _):
        for k in range(KB):
            pltpu.make_async_copy(xs_hbm.at[idx_s[k]], rows[k], gsem[k]).wait()
            pltpu.sync_copy(rows[k], acc.at[idx_d[k]], add=True)
            load_and_gather(k, (g + 1) * KB + k)
        return 0

    lax.fori_loop(0, NGRP - 1, group, 0)
    for k in range(KB):
        pltpu.make_async_copy(xs_hbm.at[idx_s[k]], rows[k], gsem[k]).wait()
        pltpu.sync_copy(rows[k], acc.at[idx_d[k]], add=True)
    plsc.subcore_barrier()

    def obody(k, _):
        b = s + NS * k
        pltpu.sync_copy(acc.at[pl.ds(b * RB, RB)],
                        out_hbm.at[pl.ds(c * N + b * RB, RB)])
        return 0

    lax.fori_loop(0, nblk, obody, 0)


def _sc_spmm(xs, src, dst, zeros_nd):
    k = pl.kernel(
        _spmm_kernel,
        out_type=jax.ShapeDtypeStruct((NC * N, D), jnp.float32),
        mesh=plsc.VectorSubcoreMesh(**_MESH),
        scratch_types=(
            [pltpu.VMEM((C,), jnp.int32)] * KB
            + [pltpu.VMEM((C,), jnp.int32)] * KB
            + [pltpu.VMEM((C, D), jnp.float32)] * KB
            + [pltpu.SemaphoreType.DMA] * KB
            + [pltpu.VMEM_SHARED((N, D), jnp.float32)]
        ),
    )
    return k(xs, src, dst, zeros_nd)


NB = 1000
NBLKS = N // NB


def _tc_a_body(x_ref, w_ref, degp_ref, xs_ref, dv_ref):
    p = degp_ref[...]
    deg = p[0, :, 0:1] + p[1, :, 0:1] + 1.0
    dinv = lax.rsqrt(deg)
    xw = jnp.dot(x_ref[...], w_ref[...], preferred_element_type=jnp.float32)
    xs_ref[...] = xw * dinv
    dv_ref[...] = jnp.broadcast_to(dinv, (NB, 16))


def _tc_a(x, Wc0, degp):
    return pl.pallas_call(
        _tc_a_body,
        grid=(NBLKS,),
        in_specs=[
            pl.BlockSpec((NB, D), lambda i: (i, 0)),
            pl.BlockSpec((D, D), lambda i: (0, 0)),
            pl.BlockSpec((NC, NB, D), lambda i: (0, i, 0)),
        ],
        out_specs=[
            pl.BlockSpec((NB, D), lambda i: (i, 0)),
            pl.BlockSpec((NB, 16), lambda i: (i, 0)),
        ],
        out_shape=[
            jax.ShapeDtypeStruct((N, D), jnp.float32),
            jax.ShapeDtypeStruct((N, 16), jnp.float32),
        ],
    )(x, Wc0, degp)


def _tc_b1_body(qp_ref, xs_ref, dv_ref, b_ref, t_ref, st_ref):
    i = pl.program_id(0)
    dinv = dv_ref[...][:, 0:1]
    pre = qp_ref[0] + qp_ref[1] + xs_ref[...]
    t = jnp.maximum(pre * dinv + b_ref[...], 0.0)
    t_ref[...] = t

    @pl.when(i == 0)
    def _():
        st_ref[...] = jnp.zeros_like(st_ref)

    st_ref[0:1, :] += jnp.sum(t, axis=0, keepdims=True)
    st_ref[1:2, :] += jnp.sum(t * t, axis=0, keepdims=True)


def _tc_b1(qp, xs0, dinvb, bc0):
    return pl.pallas_call(
        _tc_b1_body,
        grid=(NBLKS,),
        in_specs=[
            pl.BlockSpec((NC, NB, D), lambda i: (0, i, 0)),
            pl.BlockSpec((NB, D), lambda i: (i, 0)),
            pl.BlockSpec((NB, 16), lambda i: (i, 0)),
            pl.BlockSpec((1, D), lambda i: (0, 0)),
        ],
        out_specs=[
            pl.BlockSpec((NB, D), lambda i: (i, 0)),
            pl.BlockSpec((8, D), lambda i: (0, 0)),
        ],
        out_shape=[
            jax.ShapeDtypeStruct((N, D), jnp.float32),
            jax.ShapeDtypeStruct((8, D), jnp.float32),
        ],
    )(qp, xs0, dinvb, bc0)


def _tc_b2_body(t_ref, st_ref, w_ref, dv_ref, g_ref, be_ref, xs_ref):
    dinv = dv_ref[...][:, 0:1]
    m = st_ref[0:1, :] / N
    v = st_ref[1:2, :] / N - m * m
    sc = g_ref[...] * lax.rsqrt(v + EPS)
    h = (t_ref[...] - m) * sc + be_ref[...]
    xw = jnp.dot(h, w_ref[...], preferred_element_type=jnp.float32)
    xs_ref[...] = xw * dinv


def _tc_b2(t1, stats1, Wc1, dinvb, g0, be0):
    return pl.pallas_call(
        _tc_b2_body,
        grid=(NBLKS,),
        in_specs=[
            pl.BlockSpec((NB, D), lambda i: (i, 0)),
            pl.BlockSpec((8, D), lambda i: (0, 0)),
            pl.BlockSpec((D, D), lambda i: (0, 0)),
            pl.BlockSpec((NB, 16), lambda i: (i, 0)),
            pl.BlockSpec((1, D), lambda i: (0, 0)),
            pl.BlockSpec((1, D), lambda i: (0, 0)),
        ],
        out_specs=pl.BlockSpec((NB, D), lambda i: (i, 0)),
        out_shape=jax.ShapeDtypeStruct((N, D), jnp.float32),
    )(t1, stats1, Wc1, dinvb, g0, be0)


def _tc_c1_body(rp_ref, xs_ref, dv_ref, b_ref, batch_ref,
                st_ref, pool_ref, cnt_ref):
    i = pl.program_id(0)
    dinv = dv_ref[...][:, 0:1]
    pre = rp_ref[0] + rp_ref[1] + xs_ref[...]
    t = jnp.maximum(pre * dinv + b_ref[...], 0.0)

    @pl.when(i == 0)
    def _():
        st_ref[...] = jnp.zeros_like(st_ref)
        pool_ref[...] = jnp.zeros_like(pool_ref)
        cnt_ref[...] = jnp.zeros_like(cnt_ref)

    st_ref[0:1, :] += jnp.sum(t, axis=0, keepdims=True)
    st_ref[1:2, :] += jnp.sum(t * t, axis=0, keepdims=True)

    gid = lax.broadcasted_iota(jnp.int32, (1, G), 1)
    oh = (batch_ref[...] == gid).astype(jnp.float32)
    dn = (((0,), (0,)), ((), ()))
    pool_ref[...] += lax.dot_general(oh, t, dn,
                                     precision=lax.Precision.HIGHEST,
                                     preferred_element_type=jnp.float32)
    cnt_ref[...] += lax.dot_general(oh, jnp.ones_like(t), dn,
                                    precision=lax.Precision.HIGHEST,
                                    preferred_element_type=jnp.float32)


def _tc_c1(rp, xs1, dinvb, bc1, batch2d):
    return pl.pallas_call(
        _tc_c1_body,
        grid=(NBLKS,),
        in_specs=[
            pl.BlockSpec((NC, NB, D), lambda i: (0, i, 0)),
            pl.BlockSpec((NB, D), lambda i: (i, 0)),
            pl.BlockSpec((NB, 16), lambda i: (i, 0)),
            pl.BlockSpec((1, D), lambda i: (0, 0)),
            pl.BlockSpec((NB, 1), lambda i: (i, 0)),
        ],
        out_specs=[
            pl.BlockSpec((8, D), lambda i: (0, 0)),
            pl.BlockSpec((G, D), lambda i: (0, 0)),
            pl.BlockSpec((G, D), lambda i: (0, 0)),
        ],
        out_shape=[
            jax.ShapeDtypeStruct((8, D), jnp.float32),
            jax.ShapeDtypeStruct((G, D), jnp.float32),
            jax.ShapeDtypeStruct((G, D), jnp.float32),
        ],
    )(rp, xs1, dinvb, bc1, batch2d)


def _bn_rows(y, g, be):
    m = jnp.mean(y, axis=0, keepdims=True)
    v = jnp.mean(y * y, axis=0, keepdims=True) - m * m
    return (y - m) * lax.rsqrt(v + EPS) * g + be


def _tc_c2_body(st_ref, pool_ref, cnt_ref, g1_ref, be1_ref,
                wh0_ref, bh0_ref, gh0_ref, beh0_ref,
                wh1_ref, bh1_ref, gh1_ref, beh1_ref,
                wo_ref, bo_ref, out_ref):
    m2 = st_ref[0:1, :] / N
    v2 = st_ref[1:2, :] / N - m2 * m2
    s2 = g1_ref[...] * lax.rsqrt(v2 + EPS)
    cnt = cnt_ref[...]
    ph = (pool_ref[...] - cnt * m2) * s2 + cnt * be1_ref[...]
    y = jnp.maximum(jnp.dot(ph, wh0_ref[...],
                            preferred_element_type=jnp.float32)
                    + bh0_ref[...], 0.0)
    y = _bn_rows(y, gh0_ref[...], beh0_ref[...])
    y = jnp.maximum(jnp.dot(y, wh1_ref[...],
                            preferred_element_type=jnp.float32)
                    + bh1_ref[...], 0.0)
    y = _bn_rows(y, gh1_ref[...], beh1_ref[...])
    out_ref[...] = jnp.dot(y, wo_ref[...],
                           preferred_element_type=jnp.float32) + bo_ref[...]


def _tc_c2(stats2, pool, cnt, g1, be1, Wh0, bh0, gh0, beh0,
           Wh1, bh1, gh1, beh1, Wo, bo):
    row = pl.BlockSpec((1, D), lambda: (0, 0))
    return pl.pallas_call(
        _tc_c2_body,
        in_specs=[
            pl.BlockSpec((8, D), lambda: (0, 0)),
            pl.BlockSpec((G, D), lambda: (0, 0)),
            pl.BlockSpec((G, D), lambda: (0, 0)),
            row, row,
            pl.BlockSpec((D, D), lambda: (0, 0)), row, row, row,
            pl.BlockSpec((D, D), lambda: (0, 0)), row, row, row,
            pl.BlockSpec((D, 1), lambda: (0, 0)),
            pl.BlockSpec((1, 1), lambda: (0, 0)),
        ],
        out_specs=pl.BlockSpec((G, 1), lambda: (0, 0)),
        out_shape=jax.ShapeDtypeStruct((G, 1), jnp.float32),
    )(stats2, pool, cnt, g1, be1, Wh0, bh0, gh0, beh0,
      Wh1, bh1, gh1, beh1, Wo, bo)


def kernel(x, edge_index, batch, Wc0, bc0, g0, be0, Wc1, bc1, g1, be1,
           Wh0, bh0, gh0, beh0, Wh1, bh1, gh1, beh1, Wo, bo):
    pad = EP - E
    src = jnp.concatenate(
        [edge_index[0], jnp.full((pad,), N, jnp.int32)])
    dst = jnp.concatenate(
        [edge_index[1], jnp.arange(pad, dtype=jnp.int32) % N])
    dst_deg = jnp.concatenate(
        [edge_index[1],
         N + (jnp.arange(pad, dtype=jnp.int32) % PAD_ROWS)])
    zpad = jnp.zeros((8, D), jnp.float32)
    batch2d = batch.reshape(N, 1)
    bc0r = bc0.reshape(1, D)
    bc1r = bc1.reshape(1, D)
    g0r, be0r = g0.reshape(1, D), be0.reshape(1, D)
    g1r, be1r = g1.reshape(1, D), be1.reshape(1, D)
    bh0r, gh0r, beh0r = bh0.reshape(1, D), gh0.reshape(1, D), beh0.reshape(1, D)
    bh1r, gh1r, beh1r = bh1.reshape(1, D), gh1.reshape(1, D), beh1.reshape(1, D)
    bor = bo.reshape(1, 1)

    zeros_nd = jnp.zeros((N, D), jnp.float32)
    ones_c = jnp.ones((C, D), jnp.float32)

    degp = _sc_degree(dst_deg, ones_c, zeros_nd).reshape(NC, N, D)
    xs0, dinvb = _tc_a(x, Wc0, degp)
    xs0p = jnp.concatenate([xs0, zpad])
    qp = _sc_spmm(xs0p, src, dst, zeros_nd).reshape(NC, N, D)
    t1, stats1 = _tc_b1(qp, xs0, dinvb, bc0r)
    xs1 = _tc_b2(t1, stats1, Wc1, dinvb, g0r, be0r)
    xs1p = jnp.concatenate([xs1, zpad])
    rp = _sc_spmm(xs1p, src, dst, zeros_nd).reshape(NC, N, D)
    stats2, pool, cnt = _tc_c1(rp, xs1, dinvb, bc1r, batch2d)
    return _tc_c2(stats2, pool, cnt, g1r, be1r, Wh0, bh0r, gh0r, beh0r,
                  Wh1, bh1r, gh1r, beh1r, Wo, bor)


# speedup vs baseline: 1.0004x; 1.0004x over previous
import functools

import jax
import jax.numpy as jnp
from jax import lax
from jax.experimental import pallas as pl
from jax.experimental.pallas import tpu as pltpu
from jax.experimental.pallas import tpu_sc as plsc

N = 10000
E = 320000
D = 128
G = 64
EPS = 1e-5

NC, NS = 2, 16
NW = NC * NS
C = 128
PAD_ROWS = 256
NA = N + PAD_ROWS
ECH = 2592
EP = ECH * C
NCHW = ECH // NW
KB = 1
NGRP = NCHW // KB
RB = 200
NRB = N // RB
NRB_PER, NRB_REM = NRB // NS, NRB % NS

_MESH = dict(core_axis_name="c", subcore_axis_name="s")


def _deg_kernel(dst_hbm, ones_hbm, z_hbm, out_hbm, idx_d, ones_v, acc):
    c = lax.axis_index("c")
    s = lax.axis_index("s")
    wid = s * NC + c
    pltpu.sync_copy(ones_hbm, ones_v)

    nblk = jnp.where(s < NRB_REM, NRB_PER + 1, NRB_PER)

    def zbody(k, _):
        b = s + NS * k
        pltpu.sync_copy(z_hbm.at[pl.ds(b * RB, RB)], acc.at[pl.ds(b * RB, RB)])
        return 0

    lax.fori_loop(0, nblk, zbody, 0)
    plsc.subcore_barrier()

    def ebody(j, _):
        base = (wid + NW * j) * C
        pltpu.sync_copy(dst_hbm.at[pl.ds(base, C)], idx_d)
        pltpu.sync_copy(ones_v, acc.at[idx_d], add=True)
        return 0

    lax.fori_loop(0, NCHW, ebody, 0)
    plsc.subcore_barrier()

    def obody(k, _):
        b = s + NS * k
        pltpu.sync_copy(acc.at[pl.ds(b * RB, RB)],
                        out_hbm.at[pl.ds(c * N + b * RB, RB)])
        return 0

    lax.fori_loop(0, nblk, obody, 0)


def _sc_degree(dst, ones_c, zeros_nd):
    k = pl.kernel(
        _deg_kernel,
        out_type=jax.ShapeDtypeStruct((NC * N, D), jnp.float32),
        mesh=plsc.VectorSubcoreMesh(**_MESH),
        scratch_types=[
            pltpu.VMEM((C,), jnp.int32),
            pltpu.VMEM((C, D), jnp.float32),
            pltpu.VMEM_SHARED((NA, D), jnp.float32),
        ],
    )
    return k(dst, ones_c, zeros_nd)


def _spmm_kernel(xs_hbm, src_hbm, dst_hbm, z_hbm, out_hbm, *scr):
    idx_s = scr[0:KB]
    idx_d = scr[KB:2 * KB]
    rows = scr[2 * KB:3 * KB]
    gsem = scr[3 * KB:4 * KB]
    acc = scr[4 * KB]

    c = lax.axis_index("c")
    s = lax.axis_index("s")
    wid = s * NC + c

    nblk = jnp.where(s < NRB_REM, NRB_PER + 1, NRB_PER)

    def zbody(k, _):
        b = s + NS * k
        pltpu.sync_copy(z_hbm.at[pl.ds(b * RB, RB)], acc.at[pl.ds(b * RB, RB)])
        return 0

    lax.fori_loop(0, nblk, zbody, 0)
    plsc.subcore_barrier()

    def load_and_gather(k, j):
        base = (wid + NW * j) * C
        pltpu.sync_copy(src_hbm.at[pl.ds(base, C)], idx_s[k])
        pltpu.sync_copy(dst_hbm.at[pl.ds(base, C)], idx_d[k])
        pltpu.async_copy(xs_hbm.at[idx_s[k]], rows[k], gsem[k])

    def group(g, _):
        for k in range(KB):
            load_and_gather(k, g * KB + k)
            pltpu.make_async_copy(xs_hbm.at[idx_s[k]], rows[k], gsem[k]).wait()
            pltpu.sync_copy(rows[k], acc.at[idx_d[k]], add=True)
        return 0

    lax.fori_loop(0, NGRP, group, 0)
    plsc.subcore_barrier()

    def obody(k, _):
        b = s + NS * k
        pltpu.sync_copy(acc.at[pl.ds(b * RB, RB)],
                        out_hbm.at[pl.ds(c * N + b * RB, RB)])
        return 0

    lax.fori_loop(0, nblk, obody, 0)


def _sc_spmm(xs, src, dst, zeros_nd):
    k = pl.kernel(
        _spmm_kernel,
        out_type=jax.ShapeDtypeStruct((NC * N, D), jnp.float32),
        mesh=plsc.VectorSubcoreMesh(**_MESH),
        scratch_types=(
            [pltpu.VMEM((C,), jnp.int32)] * KB
            + [pltpu.VMEM((C,), jnp.int32)] * KB
            + [pltpu.VMEM((C, D), jnp.float32)] * KB
            + [pltpu.SemaphoreType.DMA] * KB
            + [pltpu.VMEM_SHARED((N, D), jnp.float32)]
        ),
    )
    return k(xs, src, dst, zeros_nd)


NB = 1000
NBLKS = N // NB


def _tc_a_body(x_ref, w_ref, degp_ref, xs_ref, dv_ref):
    p = degp_ref[...]
    deg = p[0, :, 0:1] + p[1, :, 0:1] + 1.0
    dinv = lax.rsqrt(deg)
    xw = jnp.dot(x_ref[...], w_ref[...], preferred_element_type=jnp.float32)
    xs_ref[...] = xw * dinv
    dv_ref[...] = jnp.broadcast_to(dinv, (NB, 16))


def _tc_a(x, Wc0, degp):
    return pl.pallas_call(
        _tc_a_body,
        grid=(NBLKS,),
        in_specs=[
            pl.BlockSpec((NB, D), lambda i: (i, 0)),
            pl.BlockSpec((D, D), lambda i: (0, 0)),
            pl.BlockSpec((NC, NB, D), lambda i: (0, i, 0)),
        ],
        out_specs=[
            pl.BlockSpec((NB, D), lambda i: (i, 0)),
            pl.BlockSpec((NB, 16), lambda i: (i, 0)),
        ],
        out_shape=[
            jax.ShapeDtypeStruct((N, D), jnp.float32),
            jax.ShapeDtypeStruct((N, 16), jnp.float32),
        ],
    )(x, Wc0, degp)


def _tc_b1_body(qp_ref, xs_ref, dv_ref, b_ref, t_ref, st_ref):
    i = pl.program_id(0)
    dinv = dv_ref[...][:, 0:1]
    pre = qp_ref[0] + qp_ref[1] + xs_ref[...]
    t = jnp.maximum(pre * dinv + b_ref[...], 0.0)
    t_ref[...] = t

    @pl.when(i == 0)
    def _():
        st_ref[...] = jnp.zeros_like(st_ref)

    st_ref[0:1, :] += jnp.sum(t, axis=0, keepdims=True)
    st_ref[1:2, :] += jnp.sum(t * t, axis=0, keepdims=True)


def _tc_b1(qp, xs0, dinvb, bc0):
    return pl.pallas_call(
        _tc_b1_body,
        grid=(NBLKS,),
        in_specs=[
            pl.BlockSpec((NC, NB, D), lambda i: (0, i, 0)),
            pl.BlockSpec((NB, D), lambda i: (i, 0)),
            pl.BlockSpec((NB, 16), lambda i: (i, 0)),
            pl.BlockSpec((1, D), lambda i: (0, 0)),
        ],
        out_specs=[
            pl.BlockSpec((NB, D), lambda i: (i, 0)),
            pl.BlockSpec((8, D), lambda i: (0, 0)),
        ],
        out_shape=[
            jax.ShapeDtypeStruct((N, D), jnp.float32),
            jax.ShapeDtypeStruct((8, D), jnp.float32),
        ],
    )(qp, xs0, dinvb, bc0)


def _tc_b2_body(t_ref, st_ref, w_ref, dv_ref, g_ref, be_ref, xs_ref):
    dinv = dv_ref[...][:, 0:1]
    m = st_ref[0:1, :] / N
    v = st_ref[1:2, :] / N - m * m
    sc = g_ref[...] * lax.rsqrt(v + EPS)
    h = (t_ref[...] - m) * sc + be_ref[...]
    xw = jnp.dot(h, w_ref[...], preferred_element_type=jnp.float32)
    xs_ref[...] = xw * dinv


def _tc_b2(t1, stats1, Wc1, dinvb, g0, be0):
    return pl.pallas_call(
        _tc_b2_body,
        grid=(NBLKS,),
        in_specs=[
            pl.BlockSpec((NB, D), lambda i: (i, 0)),
            pl.BlockSpec((8, D), lambda i: (0, 0)),
            pl.BlockSpec((D, D), lambda i: (0, 0)),
            pl.BlockSpec((NB, 16), lambda i: (i, 0)),
            pl.BlockSpec((1, D), lambda i: (0, 0)),
            pl.BlockSpec((1, D), lambda i: (0, 0)),
        ],
        out_specs=pl.BlockSpec((NB, D), lambda i: (i, 0)),
        out_shape=jax.ShapeDtypeStruct((N, D), jnp.float32),
    )(t1, stats1, Wc1, dinvb, g0, be0)


def _tc_c1_body(rp_ref, xs_ref, dv_ref, b_ref, batch_ref,
                st_ref, pool_ref, cnt_ref):
    i = pl.program_id(0)
    dinv = dv_ref[...][:, 0:1]
    pre = rp_ref[0] + rp_ref[1] + xs_ref[...]
    t = jnp.maximum(pre * dinv + b_ref[...], 0.0)

    @pl.when(i == 0)
    def _():
        st_ref[...] = jnp.zeros_like(st_ref)
        pool_ref[...] = jnp.zeros_like(pool_ref)
        cnt_ref[...] = jnp.zeros_like(cnt_ref)

    st_ref[0:1, :] += jnp.sum(t, axis=0, keepdims=True)
    st_ref[1:2, :] += jnp.sum(t * t, axis=0, keepdims=True)

    gid = lax.broadcasted_iota(jnp.int32, (1, G), 1)
    oh = (batch_ref[...] == gid).astype(jnp.float32)
    dn = (((0,), (0,)), ((), ()))
    pool_ref[...] += lax.dot_general(oh, t, dn,
                                     precision=lax.Precision.HIGHEST,
                                     preferred_element_type=jnp.float32)
    cnt_ref[...] += lax.dot_general(oh, jnp.ones_like(t), dn,
                                    precision=lax.Precision.HIGHEST,
                                    preferred_element_type=jnp.float32)


def _tc_c1(rp, xs1, dinvb, bc1, batch2d):
    return pl.pallas_call(
        _tc_c1_body,
        grid=(NBLKS,),
        in_specs=[
            pl.BlockSpec((NC, NB, D), lambda i: (0, i, 0)),
            pl.BlockSpec((NB, D), lambda i: (i, 0)),
            pl.BlockSpec((NB, 16), lambda i: (i, 0)),
            pl.BlockSpec((1, D), lambda i: (0, 0)),
            pl.BlockSpec((NB, 1), lambda i: (i, 0)),
        ],
        out_specs=[
            pl.BlockSpec((8, D), lambda i: (0, 0)),
            pl.BlockSpec((G, D), lambda i: (0, 0)),
            pl.BlockSpec((G, D), lambda i: (0, 0)),
        ],
        out_shape=[
            jax.ShapeDtypeStruct((8, D), jnp.float32),
            jax.ShapeDtypeStruct((G, D), jnp.float32),
            jax.ShapeDtypeStruct((G, D), jnp.float32),
        ],
    )(rp, xs1, dinvb, bc1, batch2d)


def _bn_rows(y, g, be):
    m = jnp.mean(y, axis=0, keepdims=True)
    v = jnp.mean(y * y, axis=0, keepdims=True) - m * m
    return (y - m) * lax.rsqrt(v + EPS) * g + be


def _tc_c2_body(st_ref, pool_ref, cnt_ref, g1_ref, be1_ref,
                wh0_ref, bh0_ref, gh0_ref, beh0_ref,
                wh1_ref, bh1_ref, gh1_ref, beh1_ref,
                wo_ref, bo_ref, out_ref):
    m2 = st_ref[0:1, :] / N
    v2 = st_ref[1:2, :] / N - m2 * m2
    s2 = g1_ref[...] * lax.rsqrt(v2 + EPS)
    cnt = cnt_ref[...]
    ph = (pool_ref[...] - cnt * m2) * s2 + cnt * be1_ref[...]
    y = jnp.maximum(jnp.dot(ph, wh0_ref[...],
                            preferred_element_type=jnp.float32)
                    + bh0_ref[...], 0.0)
    y = _bn_rows(y, gh0_ref[...], beh0_ref[...])
    y = jnp.maximum(jnp.dot(y, wh1_ref[...],
                            preferred_element_type=jnp.float32)
                    + bh1_ref[...], 0.0)
    y = _bn_rows(y, gh1_ref[...], beh1_ref[...])
    out_ref[...] = jnp.dot(y, wo_ref[...],
                           preferred_element_type=jnp.float32) + bo_ref[...]


def _tc_c2(stats2, pool, cnt, g1, be1, Wh0, bh0, gh0, beh0,
           Wh1, bh1, gh1, beh1, Wo, bo):
    row = pl.BlockSpec((1, D), lambda: (0, 0))
    return pl.pallas_call(
        _tc_c2_body,
        in_specs=[
            pl.BlockSpec((8, D), lambda: (0, 0)),
            pl.BlockSpec((G, D), lambda: (0, 0)),
            pl.BlockSpec((G, D), lambda: (0, 0)),
            row, row,
            pl.BlockSpec((D, D), lambda: (0, 0)), row, row, row,
            pl.BlockSpec((D, D), lambda: (0, 0)), row, row, row,
            pl.BlockSpec((D, 1), lambda: (0, 0)),
            pl.BlockSpec((1, 1), lambda: (0, 0)),
        ],
        out_specs=pl.BlockSpec((G, 1), lambda: (0, 0)),
        out_shape=jax.ShapeDtypeStruct((G, 1), jnp.float32),
    )(stats2, pool, cnt, g1, be1, Wh0, bh0, gh0, beh0,
      Wh1, bh1, gh1, beh1, Wo, bo)


def kernel(x, edge_index, batch, Wc0, bc0, g0, be0, Wc1, bc1, g1, be1,
           Wh0, bh0, gh0, beh0, Wh1, bh1, gh1, beh1, Wo, bo):
    pad = EP - E
    src = jnp.concatenate(
        [edge_index[0], jnp.full((pad,), N, jnp.int32)])
    dst = jnp.concatenate(
        [edge_index[1], jnp.arange(pad, dtype=jnp.int32) % N])
    dst_deg = jnp.concatenate(
        [edge_index[1],
         N + (jnp.arange(pad, dtype=jnp.int32) % PAD_ROWS)])
    zpad = jnp.zeros((8, D), jnp.float32)
    batch2d = batch.reshape(N, 1)
    bc0r = bc0.reshape(1, D)
    bc1r = bc1.reshape(1, D)
    g0r, be0r = g0.reshape(1, D), be0.reshape(1, D)
    g1r, be1r = g1.reshape(1, D), be1.reshape(1, D)
    bh0r, gh0r, beh0r = bh0.reshape(1, D), gh0.reshape(1, D), beh0.reshape(1, D)
    bh1r, gh1r, beh1r = bh1.reshape(1, D), gh1.reshape(1, D), beh1.reshape(1, D)
    bor = bo.reshape(1, 1)

    zeros_nd = jnp.zeros((N, D), jnp.float32)
    ones_c = jnp.ones((C, D), jnp.float32)

    degp = _sc_degree(dst_deg, ones_c, zeros_nd).reshape(NC, N, D)
    xs0, dinvb = _tc_a(x, Wc0, degp)
    xs0p = jnp.concatenate([xs0, zpad])
    qp = _sc_spmm(xs0p, src, dst, zeros_nd).reshape(NC, N, D)
    t1, stats1 = _tc_b1(qp, xs0, dinvb, bc0r)
    xs1 = _tc_b2(t1, stats1, Wc1, dinvb, g0r, be0r)
    xs1p = jnp.concatenate([xs1, zpad])
    rp = _sc_spmm(xs1p, src, dst, zeros_nd).reshape(NC, N, D)
    stats2, pool, cnt = _tc_c1(rp, xs1, dinvb, bc1r, batch2d)
    return _tc_c2(stats2, pool, cnt, g1r, be1r, Wh0, bh0r, gh0r, beh0r,
                  Wh1, bh1r, gh1r, beh1r, Wo, bor)


# speedup vs baseline: 3.1163x; 3.1150x over previous
import functools

import jax
import jax.numpy as jnp
from jax import lax
from jax.experimental import pallas as pl
from jax.experimental.pallas import tpu as pltpu
from jax.experimental.pallas import tpu_sc as plsc

N = 10000
E = 320000
D = 128
G = 64
EPS = 1e-5

NC, NS = 2, 16
NW = NC * NS
C = 128
NCHUNK = E // C
NCH_PER, NCH_REM = NCHUNK // NW, NCHUNK % NW
KB = 3
RB = 200
NRB = N // RB
NRB_PER, NRB_REM = NRB // NS, NRB % NS

_MESH = dict(core_axis_name="c", subcore_axis_name="s")


def _deg_kernel(dst_hbm, ones_hbm, z_hbm, out_hbm, idx_d, ones_v, acc):
    c = lax.axis_index("c")
    s = lax.axis_index("s")
    wid = s * NC + c
    pltpu.sync_copy(ones_hbm, ones_v)

    nblk = jnp.where(s < NRB_REM, NRB_PER + 1, NRB_PER)

    def zbody(k, _):
        b = s + NS * k
        pltpu.sync_copy(z_hbm.at[pl.ds(b * RB, RB)], acc.at[pl.ds(b * RB, RB)])
        return 0

    lax.fori_loop(0, nblk, zbody, 0)
    plsc.subcore_barrier()

    nch = jnp.where(wid < NCH_REM, NCH_PER + 1, NCH_PER)

    def ebody(j, _):
        base = (wid + NW * j) * C
        pltpu.sync_copy(dst_hbm.at[pl.ds(base, C)], idx_d)
        pltpu.sync_copy(ones_v, acc.at[idx_d], add=True)
        return 0

    lax.fori_loop(0, nch, ebody, 0)
    plsc.subcore_barrier()

    def obody(k, _):
        b = s + NS * k
        pltpu.sync_copy(acc.at[pl.ds(b * RB, RB)],
                        out_hbm.at[pl.ds(c * N + b * RB, RB)])
        return 0

    lax.fori_loop(0, nblk, obody, 0)


def _sc_degree(dst, ones_c, zeros_nd):
    k = pl.kernel(
        _deg_kernel,
        out_type=jax.ShapeDtypeStruct((NC * N, D), jnp.float32),
        mesh=plsc.VectorSubcoreMesh(**_MESH),
        scratch_types=[
            pltpu.VMEM((C,), jnp.int32),
            pltpu.VMEM((C, D), jnp.float32),
            pltpu.VMEM_SHARED((N, D), jnp.float32),
        ],
    )
    return k(dst, ones_c, zeros_nd)


def _spmm_kernel(xs_hbm, src_hbm, dst_hbm, z_hbm, out_hbm, *scr):
    idx_s = scr[0:KB]
    idx_d = scr[KB:2 * KB]
    rows = scr[2 * KB:3 * KB]
    gsem = scr[3 * KB:4 * KB]
    acc = scr[4 * KB]

    c = lax.axis_index("c")
    s = lax.axis_index("s")
    wid = s * NC + c

    nblk = jnp.where(s < NRB_REM, NRB_PER + 1, NRB_PER)

    def zbody(k, _):
        b = s + NS * k
        pltpu.sync_copy(z_hbm.at[pl.ds(b * RB, RB)], acc.at[pl.ds(b * RB, RB)])
        return 0

    lax.fori_loop(0, nblk, zbody, 0)
    plsc.subcore_barrier()

    nch = jnp.where(wid < NCH_REM, NCH_PER + 1, NCH_PER)
    ngrp = nch // KB

    def group(g, _):
        ds = []
        for k in range(KB):
            j = g * KB + k
            base = (wid + NW * j) * C
            pltpu.sync_copy(src_hbm.at[pl.ds(base, C)], idx_s[k])
            pltpu.sync_copy(dst_hbm.at[pl.ds(base, C)], idx_d[k])
            ds.append(pltpu.async_copy(xs_hbm.at[idx_s[k]], rows[k], gsem[k]))
        for k in range(KB):
            ds[k].wait()
            pltpu.sync_copy(rows[k], acc.at[idx_d[k]], add=True)
        return 0

    lax.fori_loop(0, ngrp, group, 0)

    def tail(j, _):
        base = (wid + NW * j) * C
        pltpu.sync_copy(src_hbm.at[pl.ds(base, C)], idx_s[0])
        pltpu.sync_copy(dst_hbm.at[pl.ds(base, C)], idx_d[0])
        pltpu.async_copy(xs_hbm.at[idx_s[0]], rows[0], gsem[0]).wait()
        pltpu.sync_copy(rows[0], acc.at[idx_d[0]], add=True)
        return 0

    lax.fori_loop(ngrp * KB, nch, tail, 0)
    plsc.subcore_barrier()

    def obody(k, _):
        b = s + NS * k
        pltpu.sync_copy(acc.at[pl.ds(b * RB, RB)],
                        out_hbm.at[pl.ds(c * N + b * RB, RB)])
        return 0

    lax.fori_loop(0, nblk, obody, 0)


def _sc_spmm(xs, src, dst, zeros_nd):
    k = pl.kernel(
        _spmm_kernel,
        out_type=jax.ShapeDtypeStruct((NC * N, D), jnp.float32),
        mesh=plsc.VectorSubcoreMesh(**_MESH),
        scratch_types=(
            [pltpu.VMEM((C,), jnp.int32)] * KB
            + [pltpu.VMEM((C,), jnp.int32)] * KB
            + [pltpu.VMEM((C, D), jnp.float32)] * KB
            + [pltpu.SemaphoreType.DMA] * KB
            + [pltpu.VMEM_SHARED((N, D), jnp.float32)]
        ),
    )
    return k(xs, src, dst, zeros_nd)


NB = 1000
NBLKS = N // NB


def _tc_a_body(x_ref, w_ref, degp_ref, xs_ref, dv_ref):
    p = degp_ref[...]
    deg = p[0, :, 0:1] + p[1, :, 0:1] + 1.0
    dinv = lax.rsqrt(deg)
    xw = jnp.dot(x_ref[...], w_ref[...], preferred_element_type=jnp.float32)
    xs_ref[...] = xw * dinv
    dv_ref[...] = jnp.broadcast_to(dinv, (NB, 16))


def _tc_a(x, Wc0, degp):
    return pl.pallas_call(
        _tc_a_body,
        grid=(NBLKS,),
        in_specs=[
            pl.BlockSpec((NB, D), lambda i: (i, 0)),
            pl.BlockSpec((D, D), lambda i: (0, 0)),
            pl.BlockSpec((NC, NB, D), lambda i: (0, i, 0)),
        ],
        out_specs=[
            pl.BlockSpec((NB, D), lambda i: (i, 0)),
            pl.BlockSpec((NB, 16), lambda i: (i, 0)),
        ],
        out_shape=[
            jax.ShapeDtypeStruct((N, D), jnp.float32),
            jax.ShapeDtypeStruct((N, 16), jnp.float32),
        ],
    )(x, Wc0, degp)


def _tc_b1_body(qp_ref, xs_ref, dv_ref, b_ref, t_ref, st_ref):
    i = pl.program_id(0)
    dinv = dv_ref[...][:, 0:1]
    pre = qp_ref[0] + qp_ref[1] + xs_ref[...]
    t = jnp.maximum(pre * dinv + b_ref[...], 0.0)
    t_ref[...] = t

    @pl.when(i == 0)
    def _():
        st_ref[...] = jnp.zeros_like(st_ref)

    st_ref[0:1, :] += jnp.sum(t, axis=0, keepdims=True)
    st_ref[1:2, :] += jnp.sum(t * t, axis=0, keepdims=True)


def _tc_b1(qp, xs0, dinvb, bc0):
    return pl.pallas_call(
        _tc_b1_body,
        grid=(NBLKS,),
        in_specs=[
            pl.BlockSpec((NC, NB, D), lambda i: (0, i, 0)),
            pl.BlockSpec((NB, D), lambda i: (i, 0)),
            pl.BlockSpec((NB, 16), lambda i: (i, 0)),
            pl.BlockSpec((1, D), lambda i: (0, 0)),
        ],
        out_specs=[
            pl.BlockSpec((NB, D), lambda i: (i, 0)),
            pl.BlockSpec((8, D), lambda i: (0, 0)),
        ],
        out_shape=[
            jax.ShapeDtypeStruct((N, D), jnp.float32),
            jax.ShapeDtypeStruct((8, D), jnp.float32),
        ],
    )(qp, xs0, dinvb, bc0)


def _tc_b2_body(t_ref, st_ref, w_ref, dv_ref, g_ref, be_ref, xs_ref):
    dinv = dv_ref[...][:, 0:1]
    m = st_ref[0:1, :] / N
    v = st_ref[1:2, :] / N - m * m
    sc = g_ref[...] * lax.rsqrt(v + EPS)
    h = (t_ref[...] - m) * sc + be_ref[...]
    xw = jnp.dot(h, w_ref[...], preferred_element_type=jnp.float32)
    xs_ref[...] = xw * dinv


def _tc_b2(t1, stats1, Wc1, dinvb, g0, be0):
    return pl.pallas_call(
        _tc_b2_body,
        grid=(NBLKS,),
        in_specs=[
            pl.BlockSpec((NB, D), lambda i: (i, 0)),
            pl.BlockSpec((8, D), lambda i: (0, 0)),
            pl.BlockSpec((D, D), lambda i: (0, 0)),
            pl.BlockSpec((NB, 16), lambda i: (i, 0)),
            pl.BlockSpec((1, D), lambda i: (0, 0)),
            pl.BlockSpec((1, D), lambda i: (0, 0)),
        ],
        out_specs=pl.BlockSpec((NB, D), lambda i: (i, 0)),
        out_shape=jax.ShapeDtypeStruct((N, D), jnp.float32),
    )(t1, stats1, Wc1, dinvb, g0, be0)


def _tc_c1_body(rp_ref, xs_ref, dv_ref, b_ref, batch_ref,
                st_ref, pool_ref, cnt_ref):
    i = pl.program_id(0)
    dinv = dv_ref[...][:, 0:1]
    pre = rp_ref[0] + rp_ref[1] + xs_ref[...]
    t = jnp.maximum(pre * dinv + b_ref[...], 0.0)

    @pl.when(i == 0)
    def _():
        st_ref[...] = jnp.zeros_like(st_ref)
        pool_ref[...] = jnp.zeros_like(pool_ref)
        cnt_ref[...] = jnp.zeros_like(cnt_ref)

    st_ref[0:1, :] += jnp.sum(t, axis=0, keepdims=True)
    st_ref[1:2, :] += jnp.sum(t * t, axis=0, keepdims=True)

    gid = lax.broadcasted_iota(jnp.int32, (1, G), 1)
    oh = (batch_ref[...] == gid).astype(jnp.float32)
    dn = (((0,), (0,)), ((), ()))
    pool_ref[...] += lax.dot_general(oh, t, dn,
                                     precision=lax.Precision.HIGHEST,
                                     preferred_element_type=jnp.float32)
    cnt_ref[...] += lax.dot_general(oh, jnp.ones_like(t), dn,
                                    precision=lax.Precision.HIGHEST,
                                    preferred_element_type=jnp.float32)


def _tc_c1(rp, xs1, dinvb, bc1, batch2d):
    return pl.pallas_call(
        _tc_c1_body,
        grid=(NBLKS,),
        in_specs=[
            pl.BlockSpec((NC, NB, D), lambda i: (0, i, 0)),
            pl.BlockSpec((NB, D), lambda i: (i, 0)),
            pl.BlockSpec((NB, 16), lambda i: (i, 0)),
            pl.BlockSpec((1, D), lambda i: (0, 0)),
            pl.BlockSpec((NB, 1), lambda i: (i, 0)),
        ],
        out_specs=[
            pl.BlockSpec((8, D), lambda i: (0, 0)),
            pl.BlockSpec((G, D), lambda i: (0, 0)),
            pl.BlockSpec((G, D), lambda i: (0, 0)),
        ],
        out_shape=[
            jax.ShapeDtypeStruct((8, D), jnp.float32),
            jax.ShapeDtypeStruct((G, D), jnp.float32),
            jax.ShapeDtypeStruct((G, D), jnp.float32),
        ],
    )(rp, xs1, dinvb, bc1, batch2d)


def _bn_rows(y, g, be):
    m = jnp.mean(y, axis=0, keepdims=True)
    v = jnp.mean(y * y, axis=0, keepdims=True) - m * m
    return (y - m) * lax.rsqrt(v + EPS) * g + be


def _tc_c2_body(st_ref, pool_ref, cnt_ref, g1_ref, be1_ref,
                wh0_ref, bh0_ref, gh0_ref, beh0_ref,
                wh1_ref, bh1_ref, gh1_ref, beh1_ref,
                wo_ref, bo_ref, out_ref):
    m2 = st_ref[0:1, :] / N
    v2 = st_ref[1:2, :] / N - m2 * m2
    s2 = g1_ref[...] * lax.rsqrt(v2 + EPS)
    cnt = cnt_ref[...]
    ph = (pool_ref[...] - cnt * m2) * s2 + cnt * be1_ref[...]
    y = jnp.maximum(jnp.dot(ph, wh0_ref[...],
                            preferred_element_type=jnp.float32)
                    + bh0_ref[...], 0.0)
    y = _bn_rows(y, gh0_ref[...], beh0_ref[...])
    y = jnp.maximum(jnp.dot(y, wh1_ref[...],
                            preferred_element_type=jnp.float32)
                    + bh1_ref[...], 0.0)
    y = _bn_rows(y, gh1_ref[...], beh1_ref[...])
    out_ref[...] = jnp.dot(y, wo_ref[...],
                           preferred_element_type=jnp.float32) + bo_ref[...]


def _tc_c2(stats2, pool, cnt, g1, be1, Wh0, bh0, gh0, beh0,
           Wh1, bh1, gh1, beh1, Wo, bo):
    row = pl.BlockSpec((1, D), lambda: (0, 0))
    return pl.pallas_call(
        _tc_c2_body,
        in_specs=[
            pl.BlockSpec((8, D), lambda: (0, 0)),
            pl.BlockSpec((G, D), lambda: (0, 0)),
            pl.BlockSpec((G, D), lambda: (0, 0)),
            row, row,
            pl.BlockSpec((D, D), lambda: (0, 0)), row, row, row,
            pl.BlockSpec((D, D), lambda: (0, 0)), row, row, row,
            pl.BlockSpec((D, 1), lambda: (0, 0)),
            pl.BlockSpec((1, 1), lambda: (0, 0)),
        ],
        out_specs=pl.BlockSpec((G, 1), lambda: (0, 0)),
        out_shape=jax.ShapeDtypeStruct((G, 1), jnp.float32),
    )(stats2, pool, cnt, g1, be1, Wh0, bh0, gh0, beh0,
      Wh1, bh1, gh1, beh1, Wo, bo)


def kernel(x, edge_index, batch, Wc0, bc0, g0, be0, Wc1, bc1, g1, be1,
           Wh0, bh0, gh0, beh0, Wh1, bh1, gh1, beh1, Wo, bo):
    src = edge_index[0]
    dst = edge_index[1]
    batch2d = batch.reshape(N, 1)
    bc0r = bc0.reshape(1, D)
    bc1r = bc1.reshape(1, D)
    g0r, be0r = g0.reshape(1, D), be0.reshape(1, D)
    g1r, be1r = g1.reshape(1, D), be1.reshape(1, D)
    bh0r, gh0r, beh0r = bh0.reshape(1, D), gh0.reshape(1, D), beh0.reshape(1, D)
    bh1r, gh1r, beh1r = bh1.reshape(1, D), gh1.reshape(1, D), beh1.reshape(1, D)
    bor = bo.reshape(1, 1)

    zeros_nd = jnp.zeros((N, D), jnp.float32)
    ones_c = jnp.ones((C, D), jnp.float32)

    degp = _sc_degree(dst, ones_c, zeros_nd).reshape(NC, N, D)
    xs0, dinvb = _tc_a(x, Wc0, degp)
    qp = _sc_spmm(xs0, src, dst, zeros_nd).reshape(NC, N, D)
    t1, stats1 = _tc_b1(qp, xs0, dinvb, bc0r)
    xs1 = _tc_b2(t1, stats1, Wc1, dinvb, g0r, be0r)
    rp = _sc_spmm(xs1, src, dst, zeros_nd).reshape(NC, N, D)
    stats2, pool, cnt = _tc_c1(rp, xs1, dinvb, bc1r, batch2d)
    return _tc_c2(stats2, pool, cnt, g1r, be1r, Wh0, bh0r, gh0r, beh0r,
                  Wh1, bh1r, gh1r, beh1r, Wo, bor)


# speedup vs baseline: 3.1700x; 1.0172x over previous
import functools

import jax
import jax.numpy as jnp
from jax import lax
from jax.experimental import pallas as pl
from jax.experimental.pallas import tpu as pltpu
from jax.experimental.pallas import tpu_sc as plsc

N = 10000
E = 320000
D = 128
G = 64
EPS = 1e-5

NC, NS = 2, 16
NW = NC * NS
C = 128
NCHUNK = E // C
NCH_PER, NCH_REM = NCHUNK // NW, NCHUNK % NW
KB = 3
RB = 200
NRB = N // RB
NRB_PER, NRB_REM = NRB // NS, NRB % NS

_MESH = dict(core_axis_name="c", subcore_axis_name="s")


def _deg_kernel(dst_hbm, ones_hbm, z_hbm, out_hbm, idx_d, ones_v, acc):
    c = lax.axis_index("c")
    s = lax.axis_index("s")
    wid = s * NC + c
    pltpu.sync_copy(ones_hbm, ones_v)

    nblk = jnp.where(s < NRB_REM, NRB_PER + 1, NRB_PER)

    def zbody(k, _):
        b = s + NS * k
        pltpu.sync_copy(z_hbm.at[pl.ds(b * RB, RB)], acc.at[pl.ds(b * RB, RB)])
        return 0

    lax.fori_loop(0, nblk, zbody, 0)
    plsc.subcore_barrier()

    nch = jnp.where(wid < NCH_REM, NCH_PER + 1, NCH_PER)

    def ebody(j, _):
        base = (wid + NW * j) * C
        pltpu.sync_copy(dst_hbm.at[pl.ds(base, C)], idx_d)
        pltpu.sync_copy(ones_v, acc.at[idx_d], add=True)
        return 0

    lax.fori_loop(0, nch, ebody, 0)
    plsc.subcore_barrier()

    def obody(k, _):
        b = s + NS * k
        pltpu.sync_copy(acc.at[pl.ds(b * RB, RB)],
                        out_hbm.at[pl.ds(c * N + b * RB, RB)])
        return 0

    lax.fori_loop(0, nblk, obody, 0)


def _sc_degree(dst, ones_c, zeros_nd):
    k = pl.kernel(
        _deg_kernel,
        out_type=jax.ShapeDtypeStruct((NC * N, D), jnp.float32),
        mesh=plsc.VectorSubcoreMesh(**_MESH),
        scratch_types=[
            pltpu.VMEM((C,), jnp.int32),
            pltpu.VMEM((C, D), jnp.float32),
            pltpu.VMEM_SHARED((N, D), jnp.float32),
        ],
    )
    return k(dst, ones_c, zeros_nd)


def _spmm_kernel(xs_hbm, src_hbm, dst_hbm, z_hbm, out_hbm, *scr):
    idx_s = scr[0:KB]
    idx_d = scr[KB:2 * KB]
    rows = scr[2 * KB:3 * KB]
    gsem = scr[3 * KB:4 * KB]
    acc = scr[4 * KB]

    c = lax.axis_index("c")
    s = lax.axis_index("s")
    wid = s * NC + c

    nblk = jnp.where(s < NRB_REM, NRB_PER + 1, NRB_PER)

    def zbody(k, _):
        b = s + NS * k
        pltpu.sync_copy(z_hbm.at[pl.ds(b * RB, RB)], acc.at[pl.ds(b * RB, RB)])
        return 0

    lax.fori_loop(0, nblk, zbody, 0)
    plsc.subcore_barrier()

    nch = jnp.where(wid < NCH_REM, NCH_PER + 1, NCH_PER)
    ngrp = nch // KB

    def load_and_gather(k, j):
        base = (wid + NW * j) * C
        pltpu.sync_copy(src_hbm.at[pl.ds(base, C)], idx_s[k])
        pltpu.sync_copy(dst_hbm.at[pl.ds(base, C)], idx_d[k])
        pltpu.async_copy(xs_hbm.at[idx_s[k]], rows[k], gsem[k])

    for k in range(KB):
        load_and_gather(k, k)

    def group(g, _):
        for k in range(KB):
            pltpu.make_async_copy(xs_hbm.at[idx_s[k]], rows[k], gsem[k]).wait()
            pltpu.sync_copy(rows[k], acc.at[idx_d[k]], add=True)
            load_and_gather(k, (g + 1) * KB + k)
        return 0

    lax.fori_loop(0, ngrp - 1, group, 0)
    for k in range(KB):
        pltpu.make_async_copy(xs_hbm.at[idx_s[k]], rows[k], gsem[k]).wait()
        pltpu.sync_copy(rows[k], acc.at[idx_d[k]], add=True)

    def tail(j, _):
        base = (wid + NW * j) * C
        pltpu.sync_copy(src_hbm.at[pl.ds(base, C)], idx_s[0])
        pltpu.sync_copy(dst_hbm.at[pl.ds(base, C)], idx_d[0])
        pltpu.async_copy(xs_hbm.at[idx_s[0]], rows[0], gsem[0]).wait()
        pltpu.sync_copy(rows[0], acc.at[idx_d[0]], add=True)
        return 0

    lax.fori_loop(ngrp * KB, nch, tail, 0)
    plsc.subcore_barrier()

    def obody(k, _):
        b = s + NS * k
        pltpu.sync_copy(acc.at[pl.ds(b * RB, RB)],
                        out_hbm.at[pl.ds(c * N + b * RB, RB)])
        return 0

    lax.fori_loop(0, nblk, obody, 0)


def _sc_spmm(xs, src, dst, zeros_nd):
    k = pl.kernel(
        _spmm_kernel,
        out_type=jax.ShapeDtypeStruct((NC * N, D), jnp.float32),
        mesh=plsc.VectorSubcoreMesh(**_MESH),
        scratch_types=(
            [pltpu.VMEM((C,), jnp.int32)] * KB
            + [pltpu.VMEM((C,), jnp.int32)] * KB
            + [pltpu.VMEM((C, D), jnp.float32)] * KB
            + [pltpu.SemaphoreType.DMA] * KB
            + [pltpu.VMEM_SHARED((N, D), jnp.float32)]
        ),
    )
    return k(xs, src, dst, zeros_nd)


NB = 1000
NBLKS = N // NB


def _tc_a_body(x_ref, w_ref, degp_ref, xs_ref, dv_ref):
    p = degp_ref[...]
    deg = p[0, :, 0:1] + p[1, :, 0:1] + 1.0
    dinv = lax.rsqrt(deg)
    xw = jnp.dot(x_ref[...], w_ref[...], preferred_element_type=jnp.float32)
    xs_ref[...] = xw * dinv
    dv_ref[...] = jnp.broadcast_to(dinv, (NB, 16))


def _tc_a(x, Wc0, degp):
    return pl.pallas_call(
        _tc_a_body,
        grid=(NBLKS,),
        in_specs=[
            pl.BlockSpec((NB, D), lambda i: (i, 0)),
            pl.BlockSpec((D, D), lambda i: (0, 0)),
            pl.BlockSpec((NC, NB, D), lambda i: (0, i, 0)),
        ],
        out_specs=[
            pl.BlockSpec((NB, D), lambda i: (i, 0)),
            pl.BlockSpec((NB, 16), lambda i: (i, 0)),
        ],
        out_shape=[
            jax.ShapeDtypeStruct((N, D), jnp.float32),
            jax.ShapeDtypeStruct((N, 16), jnp.float32),
        ],
    )(x, Wc0, degp)


def _tc_b1_body(qp_ref, xs_ref, dv_ref, b_ref, t_ref, st_ref):
    i = pl.program_id(0)
    dinv = dv_ref[...][:, 0:1]
    pre = qp_ref[0] + qp_ref[1] + xs_ref[...]
    t = jnp.maximum(pre * dinv + b_ref[...], 0.0)
    t_ref[...] = t

    @pl.when(i == 0)
    def _():
        st_ref[...] = jnp.zeros_like(st_ref)

    st_ref[0:1, :] += jnp.sum(t, axis=0, keepdims=True)
    st_ref[1:2, :] += jnp.sum(t * t, axis=0, keepdims=True)


def _tc_b1(qp, xs0, dinvb, bc0):
    return pl.pallas_call(
        _tc_b1_body,
        grid=(NBLKS,),
        in_specs=[
            pl.BlockSpec((NC, NB, D), lambda i: (0, i, 0)),
            pl.BlockSpec((NB, D), lambda i: (i, 0)),
            pl.BlockSpec((NB, 16), lambda i: (i, 0)),
            pl.BlockSpec((1, D), lambda i: (0, 0)),
        ],
        out_specs=[
            pl.BlockSpec((NB, D), lambda i: (i, 0)),
            pl.BlockSpec((8, D), lambda i: (0, 0)),
        ],
        out_shape=[
            jax.ShapeDtypeStruct((N, D), jnp.float32),
            jax.ShapeDtypeStruct((8, D), jnp.float32),
        ],
    )(qp, xs0, dinvb, bc0)


def _tc_b2_body(t_ref, st_ref, w_ref, dv_ref, g_ref, be_ref, xs_ref):
    dinv = dv_ref[...][:, 0:1]
    m = st_ref[0:1, :] / N
    v = st_ref[1:2, :] / N - m * m
    sc = g_ref[...] * lax.rsqrt(v + EPS)
    h = (t_ref[...] - m) * sc + be_ref[...]
    xw = jnp.dot(h, w_ref[...], preferred_element_type=jnp.float32)
    xs_ref[...] = xw * dinv


def _tc_b2(t1, stats1, Wc1, dinvb, g0, be0):
    return pl.pallas_call(
        _tc_b2_body,
        grid=(NBLKS,),
        in_specs=[
            pl.BlockSpec((NB, D), lambda i: (i, 0)),
            pl.BlockSpec((8, D), lambda i: (0, 0)),
            pl.BlockSpec((D, D), lambda i: (0, 0)),
            pl.BlockSpec((NB, 16), lambda i: (i, 0)),
            pl.BlockSpec((1, D), lambda i: (0, 0)),
            pl.BlockSpec((1, D), lambda i: (0, 0)),
        ],
        out_specs=pl.BlockSpec((NB, D), lambda i: (i, 0)),
        out_shape=jax.ShapeDtypeStruct((N, D), jnp.float32),
    )(t1, stats1, Wc1, dinvb, g0, be0)


def _tc_c1_body(rp_ref, xs_ref, dv_ref, b_ref, batch_ref,
                st_ref, pool_ref, cnt_ref):
    i = pl.program_id(0)
    dinv = dv_ref[...][:, 0:1]
    pre = rp_ref[0] + rp_ref[1] + xs_ref[...]
    t = jnp.maximum(pre * dinv + b_ref[...], 0.0)

    @pl.when(i == 0)
    def _():
        st_ref[...] = jnp.zeros_like(st_ref)
        pool_ref[...] = jnp.zeros_like(pool_ref)
        cnt_ref[...] = jnp.zeros_like(cnt_ref)

    st_ref[0:1, :] += jnp.sum(t, axis=0, keepdims=True)
    st_ref[1:2, :] += jnp.sum(t * t, axis=0, keepdims=True)

    gid = lax.broadcasted_iota(jnp.int32, (1, G), 1)
    oh = (batch_ref[...] == gid).astype(jnp.float32)
    dn = (((0,), (0,)), ((), ()))
    pool_ref[...] += lax.dot_general(oh, t, dn,
                                     precision=lax.Precision.HIGHEST,
                                     preferred_element_type=jnp.float32)
    cnt_ref[...] += lax.dot_general(oh, jnp.ones_like(t), dn,
                                    precision=lax.Precision.HIGHEST,
                                    preferred_element_type=jnp.float32)


def _tc_c1(rp, xs1, dinvb, bc1, batch2d):
    return pl.pallas_call(
        _tc_c1_body,
        grid=(NBLKS,),
        in_specs=[
            pl.BlockSpec((NC, NB, D), lambda i: (0, i, 0)),
            pl.BlockSpec((NB, D), lambda i: (i, 0)),
            pl.BlockSpec((NB, 16), lambda i: (i, 0)),
            pl.BlockSpec((1, D), lambda i: (0, 0)),
            pl.BlockSpec((NB, 1), lambda i: (i, 0)),
        ],
        out_specs=[
            pl.BlockSpec((8, D), lambda i: (0, 0)),
            pl.BlockSpec((G, D), lambda i: (0, 0)),
            pl.BlockSpec((G, D), lambda i: (0, 0)),
        ],
        out_shape=[
            jax.ShapeDtypeStruct((8, D), jnp.float32),
            jax.ShapeDtypeStruct((G, D), jnp.float32),
            jax.ShapeDtypeStruct((G, D), jnp.float32),
        ],
    )(rp, xs1, dinvb, bc1, batch2d)


def _bn_rows(y, g, be):
    m = jnp.mean(y, axis=0, keepdims=True)
    v = jnp.mean(y * y, axis=0, keepdims=True) - m * m
    return (y - m) * lax.rsqrt(v + EPS) * g + be


def _tc_c2_body(st_ref, pool_ref, cnt_ref, g1_ref, be1_ref,
                wh0_ref, bh0_ref, gh0_ref, beh0_ref,
                wh1_ref, bh1_ref, gh1_ref, beh1_ref,
                wo_ref, bo_ref, out_ref):
    m2 = st_ref[0:1, :] / N
    v2 = st_ref[1:2, :] / N - m2 * m2
    s2 = g1_ref[...] * lax.rsqrt(v2 + EPS)
    cnt = cnt_ref[...]
    ph = (pool_ref[...] - cnt * m2) * s2 + cnt * be1_ref[...]
    y = jnp.maximum(jnp.dot(ph, wh0_ref[...],
                            preferred_element_type=jnp.float32)
                    + bh0_ref[...], 0.0)
    y = _bn_rows(y, gh0_ref[...], beh0_ref[...])
    y = jnp.maximum(jnp.dot(y, wh1_ref[...],
                            preferred_element_type=jnp.float32)
                    + bh1_ref[...], 0.0)
    y = _bn_rows(y, gh1_ref[...], beh1_ref[...])
    out_ref[...] = jnp.dot(y, wo_ref[...],
                           preferred_element_type=jnp.float32) + bo_ref[...]


def _tc_c2(stats2, pool, cnt, g1, be1, Wh0, bh0, gh0, beh0,
           Wh1, bh1, gh1, beh1, Wo, bo):
    row = pl.BlockSpec((1, D), lambda: (0, 0))
    return pl.pallas_call(
        _tc_c2_body,
        in_specs=[
            pl.BlockSpec((8, D), lambda: (0, 0)),
            pl.BlockSpec((G, D), lambda: (0, 0)),
            pl.BlockSpec((G, D), lambda: (0, 0)),
            row, row,
            pl.BlockSpec((D, D), lambda: (0, 0)), row, row, row,
            pl.BlockSpec((D, D), lambda: (0, 0)), row, row, row,
            pl.BlockSpec((D, 1), lambda: (0, 0)),
            pl.BlockSpec((1, 1), lambda: (0, 0)),
        ],
        out_specs=pl.BlockSpec((G, 1), lambda: (0, 0)),
        out_shape=jax.ShapeDtypeStruct((G, 1), jnp.float32),
    )(stats2, pool, cnt, g1, be1, Wh0, bh0, gh0, beh0,
      Wh1, bh1, gh1, beh1, Wo, bo)


def kernel(x, edge_index, batch, Wc0, bc0, g0, be0, Wc1, bc1, g1, be1,
           Wh0, bh0, gh0, beh0, Wh1, bh1, gh1, beh1, Wo, bo):
    src = edge_index[0]
    dst = edge_index[1]
    batch2d = batch.reshape(N, 1)
    bc0r = bc0.reshape(1, D)
    bc1r = bc1.reshape(1, D)
    g0r, be0r = g0.reshape(1, D), be0.reshape(1, D)
    g1r, be1r = g1.reshape(1, D), be1.reshape(1, D)
    bh0r, gh0r, beh0r = bh0.reshape(1, D), gh0.reshape(1, D), beh0.reshape(1, D)
    bh1r, gh1r, beh1r = bh1.reshape(1, D), gh1.reshape(1, D), beh1.reshape(1, D)
    bor = bo.reshape(1, 1)

    zeros_nd = jnp.zeros((N, D), jnp.float32)
    ones_c = jnp.ones((C, D), jnp.float32)

    degp = _sc_degree(dst, ones_c, zeros_nd).reshape(NC, N, D)
    xs0, dinvb = _tc_a(x, Wc0, degp)
    qp = _sc_spmm(xs0, src, dst, zeros_nd).reshape(NC, N, D)
    t1, stats1 = _tc_b1(qp, xs0, dinvb, bc0r)
    xs1 = _tc_b2(t1, stats1, Wc1, dinvb, g0r, be0r)
    rp = _sc_spmm(xs1, src, dst, zeros_nd).reshape(NC, N, D)
    stats2, pool, cnt = _tc_c1(rp, xs1, dinvb, bc1r, batch2d)
    return _tc_c2(stats2, pool, cnt, g1r, be1r, Wh0, bh0r, gh0r, beh0r,
                  Wh1, bh1r, gh1r, beh1r, Wo, bor)
